# baseline (speedup 1.0000x reference)
import functools

import jax
import jax.numpy as jnp
from jax import lax
from jax.experimental import pallas as pl
from jax.experimental.pallas import tpu as pltpu
from jax.experimental.pallas import tpu_sc as plsc

N = 10000
E = 320000
D = 128
HEADS = 8
HID = 16
NC, NS, L = 2, 16, 16
NW = NC * NS
NPAD = 10240
RPW = NPAD // NW
ACC_ROWS = RPW + 8
EDG = E + N
SCHUNK = 4400
NSTAGE = EDG // SCHUNK
VECS = SCHUNK // L
B = 32
EPS_DIV = 1e-16
EPS_LN = 1e-5


def _prep_body(xp_ref, w_ref, ms_ref, md_ref, h_ref, ss_ref, sd_ref):
    h = jnp.dot(xp_ref[...], w_ref[...], preferred_element_type=jnp.float32)
    h_ref[...] = h
    ss_ref[...] = jnp.dot(h, ms_ref[...], preferred_element_type=jnp.float32,
                          precision=lax.Precision.HIGHEST)
    sd_ref[...] = jnp.dot(h, md_ref[...], preferred_element_type=jnp.float32,
                          precision=lax.Precision.HIGHEST)


_prep = pl.pallas_call(
    _prep_body,
    out_shape=[
        jax.ShapeDtypeStruct((NPAD, D), jnp.float32),
        jax.ShapeDtypeStruct((NPAD, D), jnp.float32),
        jax.ShapeDtypeStruct((NPAD, D), jnp.float32),
    ],
)


def _combine(num_ref, den_ref, b_ref, g_ref, be_ref):
    num = num_ref[...]
    den = den_ref[...]
    deninv = 1.0 / (den[:, :HEADS] + EPS_DIV)
    expand = jnp.where(
        lax.broadcasted_iota(jnp.int32, (HEADS, D), 1) // HID
        == lax.broadcasted_iota(jnp.int32, (HEADS, D), 0),
        1.0, 0.0)
    x1 = num * jnp.dot(deninv, expand, preferred_element_type=jnp.float32,
                       precision=lax.Precision.HIGHEST)
    x1 = x1 + b_ref[...]
    valid = lax.broadcasted_iota(jnp.int32, (NPAD, D), 0) < N
    denom_n = float(N * D)
    mean = jnp.sum(jnp.where(valid, x1, 0.0)) / denom_n
    xc = jnp.where(valid, x1 - mean, 0.0)
    var = jnp.sum(xc * xc) / denom_n
    y = xc * lax.rsqrt(var + EPS_LN) * g_ref[...] + be_ref[...]
    return jnp.maximum(y, 0.0)


def _mid_body(num_ref, den_ref, b_ref, g_ref, be_ref, w_ref, ms_ref, md_ref,
              h_ref, ss_ref, sd_ref):
    x2 = _combine(num_ref, den_ref, b_ref, g_ref, be_ref)
    h = jnp.dot(x2, w_ref[...], preferred_element_type=jnp.float32)
    h_ref[...] = h
    ss_ref[...] = jnp.dot(h, ms_ref[...], preferred_element_type=jnp.float32,
                          precision=lax.Precision.HIGHEST)
    sd_ref[...] = jnp.dot(h, md_ref[...], preferred_element_type=jnp.float32,
                          precision=lax.Precision.HIGHEST)


_mid = pl.pallas_call(
    _mid_body,
    out_shape=[
        jax.ShapeDtypeStruct((NPAD, D), jnp.float32),
        jax.ShapeDtypeStruct((NPAD, D), jnp.float32),
        jax.ShapeDtypeStruct((NPAD, D), jnp.float32),
    ],
)


def _final_body(num_ref, den_ref, b_ref, g_ref, be_ref, fcw_ref, fcb_ref, o_ref):
    x3 = _combine(num_ref, den_ref, b_ref, g_ref, be_ref)
    o_ref[...] = (jnp.dot(x3, fcw_ref[...], preferred_element_type=jnp.float32)
                  + fcb_ref[...])


_final = pl.pallas_call(
    _final_body,
    out_shape=jax.ShapeDtypeStruct((NPAD, 1), jnp.float32),
)


_mesh = plsc.VectorSubcoreMesh(core_axis_name="c", subcore_axis_name="s")


@functools.partial(
    pl.kernel,
    out_type=[
        jax.ShapeDtypeStruct((NPAD, D), jnp.float32),
        jax.ShapeDtypeStruct((NPAD, L), jnp.float32),
    ],
    mesh=_mesh,
    scratch_types=[
        pltpu.VMEM((SCHUNK,), jnp.int32),
        pltpu.VMEM((SCHUNK,), jnp.int32),
        pltpu.VMEM((SCHUNK + 96,), jnp.int32),
        pltpu.VMEM((B,), jnp.int32),
        pltpu.VMEM((B,), jnp.int32),
        pltpu.VMEM((B + L,), jnp.int32),
        pltpu.VMEM((B, D), jnp.float32),
        pltpu.VMEM((B, D), jnp.float32),
        pltpu.VMEM((B, D), jnp.float32),
        pltpu.VMEM((B, L), jnp.float32),
        pltpu.VMEM((ACC_ROWS, D), jnp.float32),
        pltpu.VMEM((ACC_ROWS, L), jnp.float32),
        pltpu.SemaphoreType.DMA,
        pltpu.SemaphoreType.DMA,
        pltpu.SemaphoreType.DMA,
    ],
)
def _edge(h_hbm, ss_hbm, sd_hbm, src_hbm, dst_hbm, num_hbm, den_hbm,
          sstage, dstage, sel_pk, srcbuf, dstbuf, relbuf,
          h_v, ss_v, sd_v, w_v, acc, dacc, sem0, sem1, sem2):
    c = lax.axis_index("c")
    s = lax.axis_index("s")
    gw = c * NS + s
    lo = gw * RPW
    hi = lo + RPW

    zero = jnp.zeros((L,), jnp.float32)
    zero_v = jnp.zeros((L,), jnp.int32)
    one_v = jnp.ones((L,), jnp.int32)

    def zacc(k, _):
        for j in range(D // L):
            acc[k, pl.ds(j * L, L)] = zero
        dacc[k, :] = zero
        return 0
    lax.fori_loop(0, ACC_ROWS, zacc, 0)

    def process_chunk(k, carry):
        cb = pl.multiple_of(k * B, B)
        for v in range(B // L):
            pv = sel_pk[pl.ds(cb + v * L, L)]
            sv = jnp.right_shift(pv, 14)
            dv = jnp.bitwise_and(pv, 16383)
            srcbuf[pl.ds(v * L, L)] = sv
            dstbuf[pl.ds(v * L, L)] = dv
            relbuf[pl.ds(v * L, L)] = dv - lo
        cp_h = pltpu.async_copy(h_hbm.at[srcbuf], h_v, sem0)
        cp_s = pltpu.async_copy(ss_hbm.at[srcbuf], ss_v, sem1)
        cp_d = pltpu.async_copy(sd_hbm.at[dstbuf], sd_v, sem2)
        cp_s.wait()
        cp_d.wait()

        def wbody(b, _):
            e = ss_v[b, pl.ds(0, L)] + sd_v[b, pl.ds(0, L)]
            e = jnp.maximum(e, 0.2 * e)
            e = jnp.minimum(jnp.maximum(e, -85.0), 88.0)
            y = e * 1.4426950408889634
            km = y + 12582912.0
            k = km - 12582912.0
            r = y - k
            p = 1.0 + r * (0.6931471805599453 + r * (0.2402265069591007
                + r * (0.05550410866482158 + r * (0.009618129107628477
                + r * 0.0013333558146428443))))
            q = jnp.ones((L,), jnp.float32)
            t = k
            for st in (64, 32, 16, 8, 4, 2, 1):
                up = t >= float(st)
                q = jnp.where(up, q * float(2.0 ** st), q)
                t = jnp.where(up, t - float(st), t)
                dn = t <= float(-st)
                q = jnp.where(dn, q * float(2.0 ** -st), q)
                t = jnp.where(dn, t + float(st), t)
            w_v[b, :] = p * q
            return 0
        lax.fori_loop(0, B, wbody, 0, unroll=2)
        cp_h.wait()

        def abody(b, _):
            rel = relbuf[pl.ds(b, L)][0]
            wvec = w_v[b, :]
            dacc[rel, :] = dacc[rel, :] + wvec
            for hh in range(HEADS):
                cs = pl.ds(hh * L, L)
                acc[rel, cs] = acc[rel, cs] + h_v[b, cs] * wvec[hh]
            return 0
        lax.fori_loop(0, B, abody, 0, unroll=2)
        return carry

    def stage(q, _):
        soff = pl.multiple_of(q * SCHUNK, 8)
        cpa = pltpu.async_copy(src_hbm.at[pl.ds(soff, SCHUNK)], sstage, sem0)
        cpb = pltpu.async_copy(dst_hbm.at[pl.ds(soff, SCHUNK)], dstage, sem1)
        cpa.wait()
        cpb.wait()

        def scan(v, nsel):
            dv = dstage[pl.ds(v * L, L)]
            sv = sstage[pl.ds(v * L, L)]
            m = (dv >= lo) & (dv < hi)
            mi = jnp.where(m, one_v, zero_v)
            pv = dv + jnp.left_shift(sv, 14)
            cnt = (((mi[0] + mi[1]) + (mi[2] + mi[3]))
                   + ((mi[4] + mi[5]) + (mi[6] + mi[7]))
                   + ((mi[8] + mi[9]) + (mi[10] + mi[11]))
                   + ((mi[12] + mi[13]) + (mi[14] + mi[15])))

            def append(ns):
                for i in range(L):
                    sel_pk[pl.ds(ns, L)] = zero_v + pv[i]
                    ns = ns + mi[i]
                return ns
            return lax.cond(cnt > 0, append, lambda ns: ns, nsel)
        nsel = lax.fori_loop(0, VECS, scan, 0, unroll=4)

        pad_pk = zero_v + (lo + RPW + (N << 14))
        for i in range(B // L):
            sel_pk[pl.ds(nsel + i * L, L)] = pad_pk
        nproc = (nsel + B - 1) // B
        lax.fori_loop(0, nproc, process_chunk, 0)
        return 0

    lax.fori_loop(0, NSTAGE, stage, 0)

    lo_al = pl.multiple_of(lo, RPW)
    for i in range(RPW // B):
        base = i * B

        def cprow(r, _):
            for j in range(D // L):
                h_v[r, pl.ds(j * L, L)] = acc[base + r, pl.ds(j * L, L)]
            w_v[r, :] = dacc[base + r, :]
            return 0
        lax.fori_loop(0, B, cprow, 0)
        pltpu.sync_copy(h_v, num_hbm.at[pl.ds(lo_al + i * B, B)])
        pltpu.sync_copy(w_v, den_hbm.at[pl.ds(lo_al + i * B, B)])


def _head_mat(a):
    flat = a.reshape(D)
    mask = (jnp.arange(D)[:, None] // HID) == jnp.arange(D)[None, :]
    return jnp.where(mask, flat[:, None], 0.0).astype(jnp.float32)


def kernel(x, edge_index, W1, a_src1, a_dst1, b1, g1, be1,
           W2, a_src2, a_dst2, b2, g2, be2, fcW, fcb):
    xp = jnp.pad(x, ((0, NPAD - N), (0, 0)))
    loops = jnp.arange(N, dtype=edge_index.dtype)
    srcp = jnp.concatenate([edge_index[0], loops]).astype(jnp.int32)
    dstp = jnp.concatenate([edge_index[1], loops]).astype(jnp.int32)

    h1, ss1, sd1 = _prep(xp, W1, _head_mat(a_src1), _head_mat(a_dst1))
    num1, den1 = _edge(h1, ss1, sd1, srcp, dstp)
    h2, ss2, sd2 = _mid(num1, den1, b1, g1, be1, W2,
                        _head_mat(a_src2), _head_mat(a_dst2))
    num2, den2 = _edge(h2, ss2, sd2, srcp, dstp)
    out = _final(num2, den2, b2, g2, be2, fcW, fcb)
    return out[:N, 0]


# speedup vs baseline: 1.0880x; 1.0880x over previous
import functools

import jax
import jax.numpy as jnp
from jax import lax
from jax.experimental import pallas as pl
from jax.experimental.pallas import tpu as pltpu
from jax.experimental.pallas import tpu_sc as plsc

N = 10000
E = 320000
D = 128
HEADS = 8
HID = 16
NC, NS, L = 2, 16, 16
NW = NC * NS
NPAD = 10240
RPW = NPAD // NW
ACC_ROWS = RPW + 8
EDG = E + N
SCHUNK = 4400
NSTAGE = EDG // SCHUNK
VECS = SCHUNK // L
B = 32
EPS_DIV = 1e-16
EPS_LN = 1e-5


def _prep_body(xp_ref, w_ref, ms_ref, md_ref, h_ref, ss_ref, sd_ref):
    h = jnp.dot(xp_ref[...], w_ref[...], preferred_element_type=jnp.float32)
    h_ref[...] = h
    ss_ref[...] = jnp.dot(h, ms_ref[...], preferred_element_type=jnp.float32,
                          precision=lax.Precision.HIGHEST)
    sd_ref[...] = jnp.dot(h, md_ref[...], preferred_element_type=jnp.float32,
                          precision=lax.Precision.HIGHEST)


_prep = pl.pallas_call(
    _prep_body,
    out_shape=[
        jax.ShapeDtypeStruct((NPAD, D), jnp.float32),
        jax.ShapeDtypeStruct((NPAD, D), jnp.float32),
        jax.ShapeDtypeStruct((NPAD, D), jnp.float32),
    ],
)


def _combine(num_ref, den_ref, b_ref, g_ref, be_ref):
    num = num_ref[...]
    den = den_ref[...]
    deninv = 1.0 / (den[:, :HEADS] + EPS_DIV)
    expand = jnp.where(
        lax.broadcasted_iota(jnp.int32, (HEADS, D), 1) // HID
        == lax.broadcasted_iota(jnp.int32, (HEADS, D), 0),
        1.0, 0.0)
    x1 = num * jnp.dot(deninv, expand, preferred_element_type=jnp.float32,
                       precision=lax.Precision.HIGHEST)
    x1 = x1 + b_ref[...]
    valid = lax.broadcasted_iota(jnp.int32, (NPAD, D), 0) < N
    denom_n = float(N * D)
    mean = jnp.sum(jnp.where(valid, x1, 0.0)) / denom_n
    xc = jnp.where(valid, x1 - mean, 0.0)
    var = jnp.sum(xc * xc) / denom_n
    y = xc * lax.rsqrt(var + EPS_LN) * g_ref[...] + be_ref[...]
    return jnp.maximum(y, 0.0)


def _mid_body(num_ref, den_ref, b_ref, g_ref, be_ref, w_ref, ms_ref, md_ref,
              h_ref, ss_ref, sd_ref):
    x2 = _combine(num_ref, den_ref, b_ref, g_ref, be_ref)
    h = jnp.dot(x2, w_ref[...], preferred_element_type=jnp.float32)
    h_ref[...] = h
    ss_ref[...] = jnp.dot(h, ms_ref[...], preferred_element_type=jnp.float32,
                          precision=lax.Precision.HIGHEST)
    sd_ref[...] = jnp.dot(h, md_ref[...], preferred_element_type=jnp.float32,
                          precision=lax.Precision.HIGHEST)


_mid = pl.pallas_call(
    _mid_body,
    out_shape=[
        jax.ShapeDtypeStruct((NPAD, D), jnp.float32),
        jax.ShapeDtypeStruct((NPAD, D), jnp.float32),
        jax.ShapeDtypeStruct((NPAD, D), jnp.float32),
    ],
)


def _final_body(num_ref, den_ref, b_ref, g_ref, be_ref, fcw_ref, fcb_ref, o_ref):
    x3 = _combine(num_ref, den_ref, b_ref, g_ref, be_ref)
    o_ref[...] = (jnp.dot(x3, fcw_ref[...], preferred_element_type=jnp.float32)
                  + fcb_ref[...])


_final = pl.pallas_call(
    _final_body,
    out_shape=jax.ShapeDtypeStruct((NPAD, 1), jnp.float32),
)


_mesh = plsc.VectorSubcoreMesh(core_axis_name="c", subcore_axis_name="s")


@functools.partial(
    pl.kernel,
    out_type=[
        jax.ShapeDtypeStruct((NPAD, D), jnp.float32),
        jax.ShapeDtypeStruct((NPAD, L), jnp.float32),
    ],
    mesh=_mesh,
    scratch_types=[
        pltpu.VMEM((SCHUNK,), jnp.int32),
        pltpu.VMEM((SCHUNK,), jnp.int32),
        pltpu.VMEM((SCHUNK + 96,), jnp.int32),
        pltpu.VMEM((B,), jnp.int32),
        pltpu.VMEM((B,), jnp.int32),
        pltpu.VMEM((B + L,), jnp.int32),
        pltpu.VMEM((B, D), jnp.float32),
        pltpu.VMEM((B, D), jnp.float32),
        pltpu.VMEM((B, D), jnp.float32),
        pltpu.VMEM((B, L), jnp.float32),
        pltpu.VMEM((ACC_ROWS, D), jnp.float32),
        pltpu.VMEM((ACC_ROWS, L), jnp.float32),
        pltpu.SemaphoreType.DMA,
        pltpu.SemaphoreType.DMA,
        pltpu.SemaphoreType.DMA,
    ],
)
def _edge(h_hbm, ss_hbm, sd_hbm, src_hbm, dst_hbm, num_hbm, den_hbm,
          sstage, dstage, sel_pk, srcbuf, dstbuf, relbuf,
          h_v, ss_v, sd_v, w_v, acc, dacc, sem0, sem1, sem2):
    c = lax.axis_index("c")
    s = lax.axis_index("s")
    gw = c * NS + s
    lo = gw * RPW
    hi = lo + RPW

    zero = jnp.zeros((L,), jnp.float32)
    zero_v = jnp.zeros((L,), jnp.int32)
    one_v = jnp.ones((L,), jnp.int32)

    def zacc(k, _):
        for j in range(D // L):
            acc[k, pl.ds(j * L, L)] = zero
        dacc[k, :] = zero
        return 0
    lax.fori_loop(0, ACC_ROWS, zacc, 0)

    def process_chunk(k, carry):
        cb = pl.multiple_of(k * B, B)
        for v in range(B // L):
            pv = sel_pk[pl.ds(cb + v * L, L)]
            sv = jnp.right_shift(pv, 14)
            dv = jnp.bitwise_and(pv, 16383)
            srcbuf[pl.ds(v * L, L)] = sv
            dstbuf[pl.ds(v * L, L)] = dv
            relbuf[pl.ds(v * L, L)] = dv - lo
        cp_h = pltpu.async_copy(h_hbm.at[srcbuf], h_v, sem0)
        cp_s = pltpu.async_copy(ss_hbm.at[srcbuf], ss_v, sem1)
        cp_d = pltpu.async_copy(sd_hbm.at[dstbuf], sd_v, sem2)
        cp_s.wait()
        cp_d.wait()

        def wbody(b, _):
            e = ss_v[b, pl.ds(0, L)] + sd_v[b, pl.ds(0, L)]
            e = jnp.maximum(e, 0.2 * e)
            w_v[b, :] = jnp.exp(e)
            return 0
        lax.fori_loop(0, B, wbody, 0, unroll=2)
        cp_h.wait()

        def abody(b, _):
            rel = relbuf[pl.ds(b, L)][0]
            wvec = w_v[b, :]
            dacc[rel, :] = dacc[rel, :] + wvec
            for hh in range(HEADS):
                cs = pl.ds(hh * L, L)
                acc[rel, cs] = acc[rel, cs] + h_v[b, cs] * wvec[hh]
            return 0
        lax.fori_loop(0, B, abody, 0, unroll=2)
        return carry

    def stage(q, _):
        soff = pl.multiple_of(q * SCHUNK, 8)
        cpa = pltpu.async_copy(src_hbm.at[pl.ds(soff, SCHUNK)], sstage, sem0)
        cpb = pltpu.async_copy(dst_hbm.at[pl.ds(soff, SCHUNK)], dstage, sem1)
        cpa.wait()
        cpb.wait()

        def scan(v, nsel):
            dv = dstage[pl.ds(v * L, L)]
            sv = sstage[pl.ds(v * L, L)]
            m = (dv >= lo) & (dv < hi)
            mi = jnp.where(m, one_v, zero_v)
            pv = dv + jnp.left_shift(sv, 14)
            cnt = (((mi[0] + mi[1]) + (mi[2] + mi[3]))
                   + ((mi[4] + mi[5]) + (mi[6] + mi[7]))
                   + ((mi[8] + mi[9]) + (mi[10] + mi[11]))
                   + ((mi[12] + mi[13]) + (mi[14] + mi[15])))

            def append(ns):
                for i in range(L):
                    sel_pk[pl.ds(ns, L)] = zero_v + pv[i]
                    ns = ns + mi[i]
                return ns
            return lax.cond(cnt > 0, append, lambda ns: ns, nsel)
        nsel = lax.fori_loop(0, VECS, scan, 0, unroll=4)

        pad_pk = zero_v + (lo + RPW + (N << 14))
        for i in range(B // L):
            sel_pk[pl.ds(nsel + i * L, L)] = pad_pk
        nproc = (nsel + B - 1) // B
        lax.fori_loop(0, nproc, process_chunk, 0)
        return 0

    lax.fori_loop(0, NSTAGE, stage, 0)

    lo_al = pl.multiple_of(lo, RPW)
    for i in range(RPW // B):
        base = i * B

        def cprow(r, _):
            for j in range(D // L):
                h_v[r, pl.ds(j * L, L)] = acc[base + r, pl.ds(j * L, L)]
            w_v[r, :] = dacc[base + r, :]
            return 0
        lax.fori_loop(0, B, cprow, 0)
        pltpu.sync_copy(h_v, num_hbm.at[pl.ds(lo_al + i * B, B)])
        pltpu.sync_copy(w_v, den_hbm.at[pl.ds(lo_al + i * B, B)])


def _head_mat(a):
    flat = a.reshape(D)
    mask = (jnp.arange(D)[:, None] // HID) == jnp.arange(D)[None, :]
    return jnp.where(mask, flat[:, None], 0.0).astype(jnp.float32)


def kernel(x, edge_index, W1, a_src1, a_dst1, b1, g1, be1,
           W2, a_src2, a_dst2, b2, g2, be2, fcW, fcb):
    xp = jnp.pad(x, ((0, NPAD - N), (0, 0)))
    loops = jnp.arange(N, dtype=edge_index.dtype)
    srcp = jnp.concatenate([edge_index[0], loops]).astype(jnp.int32)
    dstp = jnp.concatenate([edge_index[1], loops]).astype(jnp.int32)

    h1, ss1, sd1 = _prep(xp, W1, _head_mat(a_src1), _head_mat(a_dst1))
    num1, den1 = _edge(h1, ss1, sd1, srcp, dstp)
    h2, ss2, sd2 = _mid(num1, den1, b1, g1, be1, W2,
                        _head_mat(a_src2), _head_mat(a_dst2))
    num2, den2 = _edge(h2, ss2, sd2, srcp, dstp)
    out = _final(num2, den2, b2, g2, be2, fcW, fcb)
    return out[:N, 0]
